# Initial kernel scaffold; baseline (speedup 1.0000x reference)
import jax
import jax.numpy as jnp
from jax.experimental import pallas as pl


def kernel(x, W, b, decoder_bias):
    raise NotImplementedError("write your pallas kernel here")


# speedup vs baseline: 25.2039x; 25.2039x over previous
import functools

import jax
import jax.numpy as jnp
from jax import lax
from jax.experimental import pallas as pl

BATCH_BLK = 2048


def _encode_body(x_ref, w_ref, b_ref, a1t_ref):
    xblk = x_ref[...]
    z = lax.dot_general(
        w_ref[...], xblk, (((1,), (1,)), ((), ())),
        preferred_element_type=jnp.float32,
        precision=lax.Precision.DEFAULT,
    )
    z = z + b_ref[...]
    a1t_ref[...] = jnp.where(z > 0, z, 0.0)


def _encode(x, W, b2d):
    B, D = x.shape
    U = W.shape[0]
    grid = B // BATCH_BLK
    return pl.pallas_call(
        _encode_body,
        grid=(grid,),
        in_specs=[
            pl.BlockSpec((BATCH_BLK, D), lambda i: (i, 0)),
            pl.BlockSpec((U, D), lambda i: (0, 0)),
            pl.BlockSpec((U, 1), lambda i: (0, 0)),
        ],
        out_specs=pl.BlockSpec((U, BATCH_BLK), lambda i: (0, i)),
        out_shape=jax.ShapeDtypeStruct((U, B), jnp.float32),
    )(x, W, b2d)


def _select_body(kcount, a1t_ref, thr_ref):
    a = a1t_ref[...]
    U = a.shape[0]
    kf = jnp.float32(kcount)

    def step(i, t):
        bit = 30 - i
        cand = t | (jnp.int32(1) << bit)
        candf = lax.bitcast_convert_type(cand, jnp.float32)
        cnt = jnp.sum(jnp.where(a >= candf, 1.0, 0.0), axis=1, keepdims=True)
        return jnp.where(cnt >= kf, cand, t)

    t0 = jnp.zeros((U, 1), jnp.int32)
    t = lax.fori_loop(0, 31, step, t0)
    thr_ref[...] = lax.bitcast_convert_type(t, jnp.float32)


def _select(a1T, kcount):
    U = a1T.shape[0]
    return pl.pallas_call(
        functools.partial(_select_body, kcount),
        out_shape=jax.ShapeDtypeStruct((U, 1), jnp.float32),
    )(a1T)


def _decode_body(a1t_ref, w_ref, thr_ref, db_ref, out_ref):
    a = a1t_ref[...]
    am = jnp.where(a >= thr_ref[...], a, 0.0)
    out = lax.dot_general(
        am, w_ref[...], (((0,), (0,)), ((), ())),
        preferred_element_type=jnp.float32,
        precision=lax.Precision.DEFAULT,
    )
    out_ref[...] = out + db_ref[...]


def _decode(a1T, W, thr, db2d):
    U, B = a1T.shape
    D = W.shape[1]
    grid = B // BATCH_BLK
    return pl.pallas_call(
        _decode_body,
        grid=(grid,),
        in_specs=[
            pl.BlockSpec((U, BATCH_BLK), lambda i: (0, i)),
            pl.BlockSpec((U, D), lambda i: (0, 0)),
            pl.BlockSpec((U, 1), lambda i: (0, 0)),
            pl.BlockSpec((1, D), lambda i: (0, 0)),
        ],
        out_specs=pl.BlockSpec((BATCH_BLK, D), lambda i: (i, 0)),
        out_shape=jax.ShapeDtypeStruct((B, D), jnp.float32),
    )(a1T, W, thr, db2d)


def kernel(x, W, b, decoder_bias):
    B = x.shape[0]
    kcount = max(1, int(B * 0.05))
    a1T = _encode(x, W, b.reshape(-1, 1))
    thr = _select(a1T, kcount)
    return _decode(a1T, W, thr, decoder_bias.reshape(1, -1))
